# baseline (speedup 1.0000x reference)
import functools

import jax
import jax.numpy as jnp
from jax import lax
from jax.experimental import pallas as pl
from jax.experimental.pallas import tpu as pltpu
from jax.experimental.pallas import tpu_sc as plsc

N = 10000
E = 320000
D = 128
DE = 16
DH = 256

NC = 2
NS = 16
L = 16

EPW = E // (NC * NS)
K = 40
NCH = EPW // K
ZR = 40
NZCH = N // ZR


def _ab_body(x_ref, w1_ref, w2_ref, a_ref, b_ref):
    xb = x_ref[...]
    a_ref[...] = jnp.dot(xb, w1_ref[...], preferred_element_type=jnp.float32)
    b_ref[...] = jnp.dot(xb, w2_ref[...], preferred_element_type=jnp.float32)


def _node_proj(x, w1, w2):
    blk = 1000
    return pl.pallas_call(
        _ab_body,
        grid=(N // blk,),
        in_specs=[
            pl.BlockSpec((blk, D), lambda i: (i, 0)),
            pl.BlockSpec((D, DH), lambda i: (0, 0)),
            pl.BlockSpec((D, DH), lambda i: (0, 0)),
        ],
        out_specs=[
            pl.BlockSpec((blk, DH), lambda i: (i, 0)),
            pl.BlockSpec((blk, DH), lambda i: (i, 0)),
        ],
        out_shape=[
            jax.ShapeDtypeStruct((N, DH), jnp.float32),
            jax.ShapeDtypeStruct((N, DH), jnp.float32),
        ],
    )(x, w1, w2)


def _c_body(ea_ref, w3_ref, bm_ref, c_ref):
    c_ref[...] = (
        jnp.dot(ea_ref[...], w3_ref[...], preferred_element_type=jnp.float32)
        + bm_ref[...]
    )


def _edge_proj(ea, w3, bm):
    blk = 4000
    return pl.pallas_call(
        _c_body,
        grid=(E // blk,),
        in_specs=[
            pl.BlockSpec((blk, DE), lambda i: (i, 0)),
            pl.BlockSpec((DE, DH), lambda i: (0, 0)),
            pl.BlockSpec((1, DH), lambda i: (0, 0)),
        ],
        out_specs=pl.BlockSpec((blk, DH), lambda i: (i, 0)),
        out_shape=jax.ShapeDtypeStruct((E, DH), jnp.float32),
    )(ea, w3, bm)


def _edge_kernel_body(a_hbm, b_hbm, c_hbm, src_hbm, dst_hbm, out_hbm,
                      src_v, dst_v, rows_a, rows_b, rows_c, m_v, agg_sh,
                      sem_a, sem_b, sem_c):
    cid = lax.axis_index("c")
    sid = lax.axis_index("s")
    wid = cid * NS + sid
    ebase = wid * EPW

    def _zfill(r, _):
        for j in range(D // L):
            m_v[r, pl.ds(j * L, L)] = jnp.zeros((L,), jnp.float32)
        return 0
    lax.fori_loop(0, ZR, _zfill, 0)

    def _zcopy(i, _):
        c = sid + i * NS

        @pl.when(c < NZCH)
        def _():
            pltpu.sync_copy(m_v, agg_sh.at[pl.ds(c * ZR, ZR)])
        return 0
    lax.fori_loop(0, (NZCH + NS - 1) // NS, _zcopy, 0)
    plsc.subcore_barrier()

    def _chunk(c, _):
        pltpu.sync_copy(src_hbm.at[pl.ds(ebase + c * K, K)], src_v)
        pltpu.sync_copy(dst_hbm.at[pl.ds(ebase + c * K, K)], dst_v)
        da = pltpu.async_copy(a_hbm.at[src_v], rows_a, sem_a)
        db = pltpu.async_copy(b_hbm.at[dst_v], rows_b, sem_b)
        dc = pltpu.async_copy(c_hbm.at[pl.ds(ebase + c * K, K)], rows_c, sem_c)
        da.wait()
        db.wait()
        dc.wait()

        def _row(r, _):
            for j in range(D // L):
                g = (rows_a[r, pl.ds(j * L, L)]
                     + rows_b[r, pl.ds(j * L, L)]
                     + rows_c[r, pl.ds(j * L, L)])
                h = (rows_a[r, pl.ds(D + j * L, L)]
                     + rows_b[r, pl.ds(D + j * L, L)]
                     + rows_c[r, pl.ds(D + j * L, L)])
                sg = 1.0 / (1.0 + jnp.exp(-g))
                sh = 1.0 / (1.0 + jnp.exp(-h))
                m_v[r, pl.ds(j * L, L)] = sg * (h * sh)
            return 0
        lax.fori_loop(0, K, _row, 0)

        pltpu.sync_copy(m_v, agg_sh.at[dst_v], add=True)
        return 0
    lax.fori_loop(0, NCH, _chunk, 0)
    plsc.subcore_barrier()

    def _out(i, _):
        c = sid + i * NS

        @pl.when(c < NZCH)
        def _():
            pltpu.sync_copy(agg_sh.at[pl.ds(c * ZR, ZR)], m_v)
            pltpu.sync_copy(m_v, out_hbm.at[cid, pl.ds(c * ZR, ZR)])
        return 0
    lax.fori_loop(0, (NZCH + NS - 1) // NS, _out, 0)


_edge_kernel = functools.partial(
    pl.kernel,
    out_type=jax.ShapeDtypeStruct((NC, N, D), jnp.float32),
    mesh=plsc.VectorSubcoreMesh(core_axis_name="c", subcore_axis_name="s"),
    scratch_types=[
        pltpu.VMEM((K,), jnp.int32),
        pltpu.VMEM((K,), jnp.int32),
        pltpu.VMEM((K, DH), jnp.float32),
        pltpu.VMEM((K, DH), jnp.float32),
        pltpu.VMEM((K, DH), jnp.float32),
        pltpu.VMEM((ZR, D), jnp.float32),
        pltpu.VMEM_SHARED((N, D), jnp.float32),
        pltpu.SemaphoreType.DMA,
        pltpu.SemaphoreType.DMA,
        pltpu.SemaphoreType.DMA,
    ],
)


def _edge_pass(a, b, c, src, dst):
    return _edge_kernel(_edge_kernel_body)(a, b, c, src, dst)


def _post_body(agg_ref, x_ref, w_ref, b_ref, o_ref):
    s = agg_ref[0] + agg_ref[1]
    o_ref[...] = (
        x_ref[...]
        + jnp.dot(s, w_ref[...], preferred_element_type=jnp.float32)
        + b_ref[...]
    )


def _post(agg2, x, w_out, b_out):
    blk = 1000
    return pl.pallas_call(
        _post_body,
        grid=(N // blk,),
        in_specs=[
            pl.BlockSpec((NC, blk, D), lambda i: (0, i, 0)),
            pl.BlockSpec((blk, D), lambda i: (i, 0)),
            pl.BlockSpec((D, D), lambda i: (0, 0)),
            pl.BlockSpec((1, D), lambda i: (0, 0)),
        ],
        out_specs=pl.BlockSpec((blk, D), lambda i: (i, 0)),
        out_shape=jax.ShapeDtypeStruct((N, D), jnp.float32),
    )(agg2, x, w_out, b_out.reshape(1, D))


def kernel(x, edge_index, edge_attr, W_msg, b_msg, W_out, b_out):
    w1 = W_msg[:D]
    w2 = W_msg[D:2 * D]
    w3 = W_msg[2 * D:]
    src = edge_index[0].astype(jnp.int32)
    dst = edge_index[1].astype(jnp.int32)

    a, b = _node_proj(x, w1, w2)
    c = _edge_proj(edge_attr, w3, b_msg.reshape(1, DH))
    agg2 = _edge_pass(a, b, c, src, dst)
    return _post(agg2, x, W_out, b_out)


# speedup vs baseline: 1.2439x; 1.2439x over previous
import functools

import jax
import jax.numpy as jnp
from jax import lax
from jax.experimental import pallas as pl
from jax.experimental.pallas import tpu as pltpu
from jax.experimental.pallas import tpu_sc as plsc

N = 10000
E = 320000
D = 128
DE = 16
DH = 256

NC = 2
NS = 16
L = 16

EPW = E // (NC * NS)
K = 16
NCH = EPW // K
ZP = 80
NZP = N // ZP


def _ab_body(x_ref, w1_ref, w2_ref, a_ref, b_ref):
    xb = x_ref[...]
    a_ref[...] = jnp.dot(xb, w1_ref[...], preferred_element_type=jnp.float32)
    b_ref[...] = jnp.dot(xb, w2_ref[...], preferred_element_type=jnp.float32)


def _node_proj(x, w1, w2):
    blk = 1000
    return pl.pallas_call(
        _ab_body,
        grid=(N // blk,),
        in_specs=[
            pl.BlockSpec((blk, D), lambda i: (i, 0)),
            pl.BlockSpec((D, DH), lambda i: (0, 0)),
            pl.BlockSpec((D, DH), lambda i: (0, 0)),
        ],
        out_specs=[
            pl.BlockSpec((blk, DH), lambda i: (i, 0)),
            pl.BlockSpec((blk, DH), lambda i: (i, 0)),
        ],
        out_shape=[
            jax.ShapeDtypeStruct((N, DH), jnp.float32),
            jax.ShapeDtypeStruct((N, DH), jnp.float32),
        ],
    )(x, w1, w2)


def _c_body(ea_ref, w3_ref, bm_ref, c_ref):
    c_ref[...] = (
        jnp.dot(ea_ref[...], w3_ref[...], preferred_element_type=jnp.float32)
        + bm_ref[...]
    )


def _edge_proj(ea, w3, bm):
    blk = 4000
    return pl.pallas_call(
        _c_body,
        grid=(E // blk,),
        in_specs=[
            pl.BlockSpec((blk, DE), lambda i: (i, 0)),
            pl.BlockSpec((DE, DH), lambda i: (0, 0)),
            pl.BlockSpec((1, DH), lambda i: (0, 0)),
        ],
        out_specs=pl.BlockSpec((blk, DH), lambda i: (i, 0)),
        out_shape=jax.ShapeDtypeStruct((E, DH), jnp.float32),
    )(ea, w3, bm)


def _edge_kernel_body(a_hbm, b_hbm, c_hbm, src_hbm, dst_hbm, out_hbm,
                      src_v0, src_v1, dst_v0, dst_v1, dst_s0, dst_s1,
                      ra0, ra1, rb0, rb1, rc0, rc1, m0, m1, zp, agg_sh,
                      si0, si1, sg0, sg1, ss0, ss1):
    cid = lax.axis_index("c")
    sid = lax.axis_index("s")
    wid = cid * NS + sid
    ebase = wid * EPW

    src_v = (src_v0, src_v1)
    dst_v = (dst_v0, dst_v1)
    dst_s = (dst_s0, dst_s1)
    ra = (ra0, ra1)
    rb = (rb0, rb1)
    rc = (rc0, rc1)
    m = (m0, m1)
    si = (si0, si1)
    sg = (sg0, sg1)
    ss = (ss0, ss1)

    def issue_idx(c, p):
        off = ebase + c * K
        pltpu.async_copy(src_hbm.at[pl.ds(off, K)], src_v[p], si[p])
        pltpu.async_copy(dst_hbm.at[pl.ds(off, K)], dst_v[p], si[p])

    def drain_idx(p):
        pltpu.make_async_copy(src_hbm.at[pl.ds(0, K)], src_v[p], si[p]).wait()
        pltpu.make_async_copy(dst_hbm.at[pl.ds(0, K)], dst_v[p], si[p]).wait()

    def issue_gathers(c, p):
        pltpu.async_copy(a_hbm.at[src_v[p]], ra[p], sg[p])
        pltpu.async_copy(b_hbm.at[dst_v[p]], rb[p], sg[p])
        pltpu.async_copy(c_hbm.at[pl.ds(ebase + c * K, K)], rc[p], sg[p])

    def drain_gathers(p):
        pltpu.make_async_copy(a_hbm.at[pl.ds(0, K)], ra[p], sg[p]).wait()
        pltpu.make_async_copy(b_hbm.at[pl.ds(0, K)], rb[p], sg[p]).wait()
        pltpu.make_async_copy(c_hbm.at[pl.ds(0, K)], rc[p], sg[p]).wait()

    def issue_scatter(p):
        pltpu.async_copy(m[p], agg_sh.at[dst_s[p]], ss[p], add=True)

    def drain_scatter(p):
        pltpu.make_async_copy(m[p], agg_sh.at[pl.ds(0, K)], ss[p]).wait()

    def compute(p):
        rap, rbp, rcp, mp = ra[p], rb[p], rc[p], m[p]

        def _row(r, _):
            for j in range(D // L):
                g = (rap[r, pl.ds(j * L, L)]
                     + rbp[r, pl.ds(j * L, L)]
                     + rcp[r, pl.ds(j * L, L)])
                h = (rap[r, pl.ds(D + j * L, L)]
                     + rbp[r, pl.ds(D + j * L, L)]
                     + rcp[r, pl.ds(D + j * L, L)])
                sgt = 1.0 / (1.0 + jnp.exp(-g))
                sht = 1.0 / (1.0 + jnp.exp(-h))
                mp[r, pl.ds(j * L, L)] = sgt * (h * sht)
            return 0
        lax.fori_loop(0, K, _row, 0)

    def _zfill(r, _):
        for j in range(D // L):
            zp[r, pl.ds(j * L, L)] = jnp.zeros((L,), jnp.float32)
        return 0
    lax.fori_loop(0, ZP, _zfill, 0)

    def _zcopy(i, _):
        c = sid + i * NS

        @pl.when(c < NZP)
        def _():
            pltpu.sync_copy(zp, agg_sh.at[pl.ds(c * ZP, ZP)])
        return 0
    lax.fori_loop(0, (NZP + NS - 1) // NS, _zcopy, 0)
    plsc.subcore_barrier()

    issue_idx(0, 0)
    drain_idx(0)
    issue_gathers(0, 0)
    issue_idx(1, 1)

    def _half(t, c, p, q, last):
        drain_idx(q)
        issue_gathers(c + 1, q)

        @pl.when(t >= 1)
        def _():
            drain_scatter(p)
        drain_gathers(p)
        dst_s[p][pl.ds(0, L)] = dst_v[p][pl.ds(0, L)]
        if not last:
            issue_idx(c + 2, p)
        compute(p)
        issue_scatter(p)

    def _pair(t, _):
        _half(t, 2 * t, 0, 1, False)

        @pl.when(t < NCH // 2 - 1)
        def _():
            _half(t, 2 * t + 1, 1, 0, False)
        @pl.when(t == NCH // 2 - 1)
        def _():
            _half(t, 2 * t + 1, 1, 0, True)
        return 0
    lax.fori_loop(0, NCH // 2, _pair, 0)

    drain_scatter(0)
    drain_gathers(0)
    dst_s0[pl.ds(0, L)] = dst_v0[pl.ds(0, L)]
    compute(0)
    issue_scatter(0)

    drain_scatter(1)
    drain_scatter(0)
    plsc.subcore_barrier()

    def _out(i, _):
        c = sid + i * NS

        @pl.when(c < NZP)
        def _():
            pltpu.sync_copy(agg_sh.at[pl.ds(c * ZP, ZP)], zp)
            pltpu.sync_copy(zp, out_hbm.at[cid, pl.ds(c * ZP, ZP)])
        return 0
    lax.fori_loop(0, (NZP + NS - 1) // NS, _out, 0)


_edge_kernel = functools.partial(
    pl.kernel,
    out_type=jax.ShapeDtypeStruct((NC, N, D), jnp.float32),
    mesh=plsc.VectorSubcoreMesh(core_axis_name="c", subcore_axis_name="s"),
    scratch_types=(
        [pltpu.VMEM((K,), jnp.int32)] * 6
        + [pltpu.VMEM((K, DH), jnp.float32)] * 6
        + [pltpu.VMEM((K, D), jnp.float32)] * 2
        + [pltpu.VMEM((ZP, D), jnp.float32)]
        + [pltpu.VMEM_SHARED((N, D), jnp.float32)]
        + [pltpu.SemaphoreType.DMA] * 6
    ),
)


def _edge_pass(a, b, c, src, dst):
    return _edge_kernel(_edge_kernel_body)(a, b, c, src, dst)


def _post_body(agg_ref, x_ref, w_ref, b_ref, o_ref):
    s = agg_ref[0] + agg_ref[1]
    o_ref[...] = (
        x_ref[...]
        + jnp.dot(s, w_ref[...], preferred_element_type=jnp.float32)
        + b_ref[...]
    )


def _post(agg2, x, w_out, b_out):
    blk = 1000
    return pl.pallas_call(
        _post_body,
        grid=(N // blk,),
        in_specs=[
            pl.BlockSpec((NC, blk, D), lambda i: (0, i, 0)),
            pl.BlockSpec((blk, D), lambda i: (i, 0)),
            pl.BlockSpec((D, D), lambda i: (0, 0)),
            pl.BlockSpec((1, D), lambda i: (0, 0)),
        ],
        out_specs=pl.BlockSpec((blk, D), lambda i: (i, 0)),
        out_shape=jax.ShapeDtypeStruct((N, D), jnp.float32),
    )(agg2, x, w_out, b_out.reshape(1, D))


def kernel(x, edge_index, edge_attr, W_msg, b_msg, W_out, b_out):
    w1 = W_msg[:D]
    w2 = W_msg[D:2 * D]
    w3 = W_msg[2 * D:]
    src = edge_index[0].astype(jnp.int32)
    dst = edge_index[1].astype(jnp.int32)

    a, b = _node_proj(x, w1, w2)
    c = _edge_proj(edge_attr, w3, b_msg.reshape(1, DH))
    agg2 = _edge_pass(a, b, c, src, dst)
    return _post(agg2, x, W_out, b_out)


# speedup vs baseline: 3.2392x; 2.6041x over previous
import functools

import jax
import jax.numpy as jnp
from jax import lax
from jax.experimental import pallas as pl
from jax.experimental.pallas import tpu as pltpu
from jax.experimental.pallas import tpu_sc as plsc

N = 10000
E = 320000
D = 128
DE = 16
DH = 256

NC = 2
NS = 16
L = 16

EPW = E // (NC * NS)
K = 16
NCH = EPW // K
ZP = 80
NZP = N // ZP


def _ab_body(x_ref, w1_ref, w2_ref, a_ref, b_ref):
    xb = x_ref[...]
    a_ref[...] = jnp.dot(xb, w1_ref[...], preferred_element_type=jnp.float32)
    b_ref[...] = jnp.dot(xb, w2_ref[...], preferred_element_type=jnp.float32)


def _node_proj(x, w1, w2):
    blk = 1000
    return pl.pallas_call(
        _ab_body,
        grid=(N // blk,),
        in_specs=[
            pl.BlockSpec((blk, D), lambda i: (i, 0)),
            pl.BlockSpec((D, DH), lambda i: (0, 0)),
            pl.BlockSpec((D, DH), lambda i: (0, 0)),
        ],
        out_specs=[
            pl.BlockSpec((blk, DH), lambda i: (i, 0)),
            pl.BlockSpec((blk, DH), lambda i: (i, 0)),
        ],
        out_shape=[
            jax.ShapeDtypeStruct((N, DH), jnp.float32),
            jax.ShapeDtypeStruct((N, DH), jnp.float32),
        ],
    )(x, w1, w2)


def _c_body(ea_ref, w3_ref, bm_ref, c_ref):
    c_ref[...] = (
        jnp.dot(ea_ref[...], w3_ref[...], preferred_element_type=jnp.float32)
        + bm_ref[...]
    )


def _edge_proj(ea, w3, bm):
    blk = 4000
    return pl.pallas_call(
        _c_body,
        grid=(E // blk,),
        in_specs=[
            pl.BlockSpec((blk, DE), lambda i: (i, 0)),
            pl.BlockSpec((DE, DH), lambda i: (0, 0)),
            pl.BlockSpec((1, DH), lambda i: (0, 0)),
        ],
        out_specs=pl.BlockSpec((blk, DH), lambda i: (i, 0)),
        out_shape=jax.ShapeDtypeStruct((E, DH), jnp.float32),
    )(ea, w3, bm)


def _edge_kernel_body(a_hbm, b_hbm, c_hbm, src_hbm, dst_hbm, out_hbm,
                      src_v0, src_v1, dst_v0, dst_v1, dst_s0, dst_s1,
                      ra0, ra1, rb0, rb1, rc0, rc1, m0, m1, zp, agg_sh,
                      si0, si1, sg0, sg1, ss0, ss1):
    cid = lax.axis_index("c")
    sid = lax.axis_index("s")
    wid = cid * NS + sid
    ebase = wid * EPW

    src_v = (src_v0, src_v1)
    dst_v = (dst_v0, dst_v1)
    dst_s = (dst_s0, dst_s1)
    ra = (ra0, ra1)
    rb = (rb0, rb1)
    rc = (rc0, rc1)
    m = (m0, m1)
    si = (si0, si1)
    sg = (sg0, sg1)
    ss = (ss0, ss1)

    def issue_idx(c, p):
        off = ebase + c * K
        pltpu.async_copy(src_hbm.at[pl.ds(off, K)], src_v[p], si[p])
        pltpu.async_copy(dst_hbm.at[pl.ds(off, K)], dst_v[p], si[p])

    def drain_idx(p):
        pltpu.make_async_copy(src_hbm.at[pl.ds(0, K)], src_v[p], si[p]).wait()
        pltpu.make_async_copy(dst_hbm.at[pl.ds(0, K)], dst_v[p], si[p]).wait()

    def issue_gathers(c, p):
        pltpu.async_copy(a_hbm.at[src_v[p]], ra[p], sg[p])
        pltpu.async_copy(b_hbm.at[dst_v[p]], rb[p], sg[p])
        pltpu.async_copy(c_hbm.at[pl.ds(ebase + c * K, K)], rc[p], sg[p])

    def drain_gathers(p):
        pltpu.make_async_copy(a_hbm.at[pl.ds(0, K)], ra[p], sg[p]).wait()
        pltpu.make_async_copy(b_hbm.at[pl.ds(0, K)], rb[p], sg[p]).wait()
        pltpu.make_async_copy(c_hbm.at[pl.ds(0, K)], rc[p], sg[p]).wait()

    def issue_scatter(p):
        pltpu.async_copy(m[p], agg_sh.at[dst_s[p]], ss[p], add=True)

    def drain_scatter(p):
        pltpu.make_async_copy(m[p], agg_sh.at[pl.ds(0, K)], ss[p]).wait()

    def compute(p):
        rap, rbp, rcp, mp = ra[p], rb[p], rc[p], m[p]

        def _row(r, _):
            for half in range(2):
                ks = [half * 4 + j for j in range(4)]
                gs = [rap[r, pl.ds(k * L, L)]
                      + rbp[r, pl.ds(k * L, L)]
                      + rcp[r, pl.ds(k * L, L)] for k in ks]
                hs = [rap[r, pl.ds(D + k * L, L)]
                      + rbp[r, pl.ds(D + k * L, L)]
                      + rcp[r, pl.ds(D + k * L, L)] for k in ks]
                egs = [jnp.exp(-g) for g in gs]
                ehs = [jnp.exp(-h) for h in hs]
                for i, k in enumerate(ks):
                    den = (1.0 + egs[i]) * (1.0 + ehs[i])
                    mp[r, pl.ds(k * L, L)] = hs[i] / den
            return 0
        lax.fori_loop(0, K, _row, 0)

    def _zfill(r, _):
        for j in range(D // L):
            zp[r, pl.ds(j * L, L)] = jnp.zeros((L,), jnp.float32)
        return 0
    lax.fori_loop(0, ZP, _zfill, 0)

    def _zcopy(i, _):
        c = sid + i * NS

        @pl.when(c < NZP)
        def _():
            pltpu.sync_copy(zp, agg_sh.at[pl.ds(c * ZP, ZP)])
        return 0
    lax.fori_loop(0, (NZP + NS - 1) // NS, _zcopy, 0)
    plsc.subcore_barrier()

    issue_idx(0, 0)
    drain_idx(0)
    issue_gathers(0, 0)
    issue_idx(1, 1)

    def _half(t, c, p, q, last):
        drain_idx(q)
        issue_gathers(c + 1, q)

        @pl.when(t >= 1)
        def _():
            drain_scatter(p)
        drain_gathers(p)
        dst_s[p][pl.ds(0, L)] = dst_v[p][pl.ds(0, L)]
        if not last:
            issue_idx(c + 2, p)
        compute(p)
        issue_scatter(p)

    def _pair(t, _):
        _half(t, 2 * t, 0, 1, False)

        @pl.when(t < NCH // 2 - 1)
        def _():
            _half(t, 2 * t + 1, 1, 0, False)
        @pl.when(t == NCH // 2 - 1)
        def _():
            _half(t, 2 * t + 1, 1, 0, True)
        return 0
    lax.fori_loop(0, NCH // 2, _pair, 0)

    drain_scatter(0)
    drain_gathers(0)
    dst_s0[pl.ds(0, L)] = dst_v0[pl.ds(0, L)]
    compute(0)
    issue_scatter(0)

    drain_scatter(1)
    drain_scatter(0)
    plsc.subcore_barrier()

    def _out(i, _):
        c = sid + i * NS

        @pl.when(c < NZP)
        def _():
            pltpu.sync_copy(agg_sh.at[pl.ds(c * ZP, ZP)], zp)
            pltpu.sync_copy(zp, out_hbm.at[cid, pl.ds(c * ZP, ZP)])
        return 0
    lax.fori_loop(0, (NZP + NS - 1) // NS, _out, 0)


_edge_kernel = functools.partial(
    pl.kernel,
    out_type=jax.ShapeDtypeStruct((NC, N, D), jnp.float32),
    mesh=plsc.VectorSubcoreMesh(core_axis_name="c", subcore_axis_name="s"),
    scratch_types=(
        [pltpu.VMEM((K,), jnp.int32)] * 6
        + [pltpu.VMEM((K, DH), jnp.float32)] * 6
        + [pltpu.VMEM((K, D), jnp.float32)] * 2
        + [pltpu.VMEM((ZP, D), jnp.float32)]
        + [pltpu.VMEM_SHARED((N, D), jnp.float32)]
        + [pltpu.SemaphoreType.DMA] * 6
    ),
)


def _edge_pass(a, b, c, src, dst):
    return _edge_kernel(_edge_kernel_body)(a, b, c, src, dst)


def _post_body(agg_ref, x_ref, w_ref, b_ref, o_ref):
    s = agg_ref[0] + agg_ref[1]
    o_ref[...] = (
        x_ref[...]
        + jnp.dot(s, w_ref[...], preferred_element_type=jnp.float32)
        + b_ref[...]
    )


def _post(agg2, x, w_out, b_out):
    blk = 1000
    return pl.pallas_call(
        _post_body,
        grid=(N // blk,),
        in_specs=[
            pl.BlockSpec((NC, blk, D), lambda i: (0, i, 0)),
            pl.BlockSpec((blk, D), lambda i: (i, 0)),
            pl.BlockSpec((D, D), lambda i: (0, 0)),
            pl.BlockSpec((1, D), lambda i: (0, 0)),
        ],
        out_specs=pl.BlockSpec((blk, D), lambda i: (i, 0)),
        out_shape=jax.ShapeDtypeStruct((N, D), jnp.float32),
    )(agg2, x, w_out, b_out.reshape(1, D))


def kernel(x, edge_index, edge_attr, W_msg, b_msg, W_out, b_out):
    w1 = W_msg[:D]
    w2 = W_msg[D:2 * D]
    w3 = W_msg[2 * D:]
    src = edge_index[0].astype(jnp.int32)
    dst = edge_index[1].astype(jnp.int32)

    a, b = _node_proj(x, w1, w2)
    c = _edge_proj(edge_attr, w3, b_msg.reshape(1, DH))
    agg2 = _edge_pass(a, b, c, src, dst)
    return _post(agg2, x, W_out, b_out)
